# baseline (speedup 1.0000x reference)
import functools

import jax
import jax.numpy as jnp
from jax import lax
from jax.experimental import pallas as pl
from jax.experimental.pallas import tpu as pltpu
from jax.experimental.pallas import tpu_sc as plsc

B = 64
S = 512
H = 768
V = 30522
EPS = 1e-12
NC = 2
NS = 16
NW = NC * NS
SPT = S // NW
HV = H // 16


_GATHER_DNUMS = lax.GatherDimensionNumbers(
    offset_dims=(), collapsed_slice_dims=(0,), start_index_map=(0,))


def _lane_sum(x):
    lanes = lax.iota(jnp.int32, 16)
    for sh in (8, 4, 2, 1):
        idx = (lanes + sh) & 15
        x = x + lax.gather(x, idx[:, None], _GATHER_DNUMS, (1,),
                           mode=lax.GatherScatterMode.PROMISE_IN_BOUNDS)
    return x


def _rsqrt_vec(x):
    half = jnp.full((16,), 0.5, jnp.float32)
    three_half = jnp.full((16,), 1.5, jnp.float32)
    i = plsc.bitcast(x, jnp.int32)
    i = jnp.full((16,), 0x5F3759DF, jnp.int32) - lax.shift_right_arithmetic(i, 1)
    y = plsc.bitcast(i, jnp.float32)
    hx = half * x
    for _ in range(3):
        y = y * (three_half - hx * y * y)
    return y


def _tile_body(ids_hbm, word_hbm, pos_hbm, tok_hbm, g_hbm, bt_hbm, out_hbm,
               idx_v, bias_v, tok_v, gamma_v, beta_v, buf_v, sem):
    c = lax.axis_index("c")
    s_ = lax.axis_index("s")
    w = s_ * NC + c

    pltpu.sync_copy(ids_hbm.at[w], idx_v)
    pltpu.sync_copy(pos_hbm.at[pl.ds(w * SPT, SPT)], bias_v)
    pltpu.sync_copy(tok_hbm.at[0], tok_v)
    pltpu.sync_copy(g_hbm, gamma_v)
    pltpu.sync_copy(bt_hbm, beta_v)

    def _fold(sl, carry):
        for j in range(HV):
            d = pl.ds(j * 16, 16)
            bias_v[sl, d] = bias_v[sl, d] + tok_v[d]
        return carry
    lax.fori_loop(0, SPT, _fold, 0)

    one_over_h = jnp.full((16,), 1.0 / H, jnp.float32)
    eps_v = jnp.full((16,), EPS, jnp.float32)

    def _b_step(b, carry):
        pltpu.async_copy(word_hbm.at[idx_v.at[b]], buf_v, sem).wait()

        def _r_step(r, inner):
            sumv = jnp.zeros((16,), jnp.float32)
            sqv = jnp.zeros((16,), jnp.float32)
            for j in range(HV):
                d = pl.ds(j * 16, 16)
                xb = buf_v[r, d] + bias_v[r, d]
                buf_v[r, d] = xb
                sumv = sumv + xb
                sqv = sqv + xb * xb
            mean = _lane_sum(sumv) * one_over_h
            var = _lane_sum(sqv) * one_over_h - mean * mean
            istd = _rsqrt_vec(var + eps_v)
            for j in range(HV):
                d = pl.ds(j * 16, 16)
                t = (buf_v[r, d] - mean) * istd
                buf_v[r, d] = t * gamma_v[d] + beta_v[d]
            return inner
        lax.fori_loop(0, SPT, _r_step, 0)

        pltpu.sync_copy(buf_v, out_hbm.at[pl.ds(b * S + w * SPT, SPT)])
        return carry
    lax.fori_loop(0, B, _b_step, 0)


_sc_call = functools.partial(
    pl.kernel,
    out_type=jax.ShapeDtypeStruct((B * S, H), jnp.float32),
    mesh=plsc.VectorSubcoreMesh(core_axis_name="c", subcore_axis_name="s"),
    compiler_params=pltpu.CompilerParams(needs_layout_passes=False),
    scratch_types=[
        pltpu.VMEM((B, SPT), jnp.int32),
        pltpu.VMEM((SPT, H), jnp.float32),
        pltpu.VMEM((H,), jnp.float32),
        pltpu.VMEM((H,), jnp.float32),
        pltpu.VMEM((H,), jnp.float32),
        pltpu.VMEM((SPT, H), jnp.float32),
        pltpu.SemaphoreType.DMA,
    ],
)(_tile_body)


def kernel(input_ids, word_emb, pos_emb, tok_emb, gamma, beta):
    ids = input_ids.astype(jnp.int32).reshape(B, NW, SPT).transpose(1, 0, 2)
    out = _sc_call(ids, word_emb, pos_emb, tok_emb, gamma, beta)
    return out.reshape(B, S, H)
